# baseline (speedup 1.0000x reference)
import jax
import jax.numpy as jnp
from jax.experimental import pallas as pl
from jax.experimental.pallas import tpu as pltpu

_BLOCK = 256


def _fused_gnn_kernel(feat_ref, w_ref, adj_ref, out_ref, support_ref):
    @pl.when(pl.program_id(0) == 0)
    def _():
        support_ref[...] = jnp.dot(
            feat_ref[...], w_ref[...], preferred_element_type=jnp.float32
        )

    out_ref[...] = jnp.maximum(
        jnp.dot(adj_ref[...], support_ref[...], preferred_element_type=jnp.float32),
        0.0,
    )


def kernel(features, adj, weight):
    n, d_in = features.shape
    d_out = weight.shape[1]
    return pl.pallas_call(
        _fused_gnn_kernel,
        grid=(n // _BLOCK,),
        in_specs=[
            pl.BlockSpec((n, d_in), lambda i: (0, 0)),
            pl.BlockSpec((d_in, d_out), lambda i: (0, 0)),
            pl.BlockSpec((_BLOCK, n), lambda i: (i, 0)),
        ],
        out_specs=pl.BlockSpec((_BLOCK, d_out), lambda i: (i, 0)),
        out_shape=jax.ShapeDtypeStruct((n, d_out), jnp.float32),
        scratch_shapes=[pltpu.VMEM((n, d_out), jnp.float32)],
    )(features, weight, adj)


# speedup vs baseline: 1.1484x; 1.1484x over previous
import jax
import jax.numpy as jnp
from jax.experimental import pallas as pl
from jax.experimental.pallas import tpu as pltpu

_BLOCK = 512


def _fused_gnn_kernel(feat_ref, w_ref, adj_ref, out_ref, support_ref):
    @pl.when(pl.program_id(0) == 0)
    def _():
        support_ref[...] = jnp.dot(
            feat_ref[...], w_ref[...], preferred_element_type=jnp.float32
        )

    out_ref[...] = jnp.maximum(
        jnp.dot(adj_ref[...], support_ref[...], preferred_element_type=jnp.float32),
        0.0,
    )


def kernel(features, adj, weight):
    n, d_in = features.shape
    d_out = weight.shape[1]
    return pl.pallas_call(
        _fused_gnn_kernel,
        grid=(n // _BLOCK,),
        in_specs=[
            pl.BlockSpec((n, d_in), lambda i: (0, 0)),
            pl.BlockSpec((d_in, d_out), lambda i: (0, 0)),
            pl.BlockSpec((_BLOCK, n), lambda i: (i, 0)),
        ],
        out_specs=pl.BlockSpec((_BLOCK, d_out), lambda i: (i, 0)),
        out_shape=jax.ShapeDtypeStruct((n, d_out), jnp.float32),
        scratch_shapes=[pltpu.VMEM((n, d_out), jnp.float32)],
    )(features, weight, adj)
